# baseline (speedup 1.0000x reference)
import functools

import jax
import jax.numpy as jnp
from jax import lax
from jax.experimental import pallas as pl
from jax.experimental.pallas import tpu as pltpu
from jax.experimental.pallas import tpu_sc as plsc

NN = 50000
EE = 800000
EPAD = 802816
NPAD = 50016
EB = 8192
NBK = 2000
CH = 128
RPT = NPAD // 16
F32 = jnp.float32


def _fix(shape):
    return pl.BlockSpec(shape, lambda i: (0,) * len(shape))


def _ebs(f):
    return pl.BlockSpec((EB, f), lambda i: (i, 0))


def _ebs3(c, f):
    return pl.BlockSpec((c, EB, f), lambda i: (0, i, 0))


def _nbs(f):
    return pl.BlockSpec((NBK, f), lambda i: (i, 0))


def _relu(v):
    return jnp.maximum(v, 0.0)


def _mm(a, b):
    return jnp.dot(a, b, preferred_element_type=F32)


def _cat2(xi, xj):
    return jnp.concatenate([xi, xj - xi], axis=1)


def _acc_stats(i, y, st_ref):
    mask = (i * EB + lax.broadcasted_iota(jnp.int32, (EB, 1), 0)) < EE
    ym = jnp.where(mask, y, 0.0)
    blk = jnp.concatenate(
        [jnp.sum(ym, 0, keepdims=True), jnp.sum(ym * ym, 0, keepdims=True)], 0)

    @pl.when(i == 0)
    def _():
        st_ref[...] = jnp.zeros_like(st_ref)

    st_ref[...] += blk


def _ab(st, g, b, n):
    m = st[0] / n
    v = st[1] / n - m * m
    a = g / jnp.sqrt(v + 1e-5)
    return jnp.stack([a, b - m * a])


def _sc_mesh():
    return plsc.VectorSubcoreMesh(core_axis_name="c", subcore_axis_name="s")


_SC_PARAMS = pltpu.CompilerParams(use_tc_tiling_on_sc=False)


GK = 7


@functools.lru_cache(maxsize=None)
def _gather2(fa, fb, dt=jnp.float32):
    rows_w = (EPAD // CH) // 32
    ngrp = rows_w // GK

    @functools.partial(
        pl.kernel, mesh=_sc_mesh(), compiler_params=_SC_PARAMS,
        out_type=[jax.ShapeDtypeStruct((EPAD, fa), dt),
                  jax.ShapeDtypeStruct((EPAD, fb), dt)],
        scratch_types=[pltpu.VMEM((GK, CH), jnp.int32),
                       pltpu.VMEM((GK * CH, fa), dt),
                       pltpu.VMEM((GK, CH), jnp.int32),
                       pltpu.VMEM((GK * CH, fb), dt),
                       pltpu.SemaphoreType.DMA,
                       pltpu.SemaphoreType.DMA],
    )
    def k(ta, ia, tb, ib, outa, outb, iva, rva, ivb, rvb, sa, sb):
        wid = lax.axis_index("c") * 16 + lax.axis_index("s")

        def body(g, carry):
            crow = wid * rows_w + g * GK
            base = crow * CH
            pltpu.sync_copy(ia.at[pl.ds(crow, GK)], iva)
            pltpu.sync_copy(ib.at[pl.ds(crow, GK)], ivb)
            cps = []
            for b in range(GK):
                cps.append(pltpu.async_copy(
                    ta.at[iva.at[b]], rva.at[pl.ds(b * CH, CH)], sa))
                cps.append(pltpu.async_copy(
                    tb.at[ivb.at[b]], rvb.at[pl.ds(b * CH, CH)], sb))
            for cp in cps:
                cp.wait()
            pltpu.sync_copy(rva, outa.at[pl.ds(base, GK * CH)])
            pltpu.sync_copy(rvb, outb.at[pl.ds(base, GK * CH)])
            return carry

        lax.fori_loop(0, ngrp, body, 0)

    return k


@functools.lru_cache(maxsize=None)
def _scatter(feature_split, fw):
    if feature_split:
        rows_w = (EPAD // CH) // 16
    else:
        rows_w = (EPAD // CH) // 32
    ngrp = rows_w // GK

    @functools.partial(
        pl.kernel, mesh=_sc_mesh(), compiler_params=_SC_PARAMS,
        out_type=jax.ShapeDtypeStruct((2, NPAD, fw), F32),
        scratch_types=[pltpu.VMEM((GK, CH), jnp.int32),
                       pltpu.VMEM((GK * CH, fw), F32),
                       pltpu.VMEM_SHARED((NPAD, fw), F32),
                       pltpu.SemaphoreType.DMA],
    )
    def k(y, d, z, out, idx_v, row_v, acc, sem):
        cid = lax.axis_index("c")
        sid = lax.axis_index("s")
        pltpu.sync_copy(z, acc.at[pl.ds(sid * RPT, RPT)])
        plsc.subcore_barrier()

        def body(g, carry):
            if feature_split:
                crow = sid * rows_w + g * GK
                pltpu.sync_copy(y.at[cid, pl.ds(crow * CH, GK * CH)], row_v)
            else:
                crow = (cid * 16 + sid) * rows_w + g * GK
                pltpu.sync_copy(y.at[0, pl.ds(crow * CH, GK * CH)], row_v)
            pltpu.sync_copy(d.at[pl.ds(crow, GK)], idx_v)
            cps = [pltpu.async_copy(row_v.at[pl.ds(b * CH, CH)],
                                    acc.at[idx_v.at[b]], sem, add=True)
                   for b in range(GK)]
            for cp in cps:
                cp.wait()
            return carry

        lax.fori_loop(0, ngrp, body, 0)
        plsc.subcore_barrier()
        pltpu.sync_copy(acc.at[pl.ds(sid * RPT, RPT)],
                        out.at[cid, pl.ds(sid * RPT, RPT)])

    return k


@functools.lru_cache(maxsize=None)
def _counts():
    rows_w = (EPAD // CH) // 32
    ngrp = rows_w // GK

    @functools.partial(
        pl.kernel, mesh=_sc_mesh(), compiler_params=_SC_PARAMS,
        out_type=jax.ShapeDtypeStruct((2, NPAD, 16), F32),
        scratch_types=[pltpu.VMEM((GK, CH), jnp.int32),
                       pltpu.VMEM((CH, 16), F32),
                       pltpu.VMEM_SHARED((NPAD, 16), F32),
                       pltpu.SemaphoreType.DMA],
    )
    def k(d, z, ones_hbm, out, idx_v, ones_v, acc, sem):
        cid = lax.axis_index("c")
        sid = lax.axis_index("s")
        pltpu.sync_copy(z, acc.at[pl.ds(sid * RPT, RPT)])
        pltpu.sync_copy(ones_hbm, ones_v)
        plsc.subcore_barrier()

        def body(g, carry):
            crow = (cid * 16 + sid) * rows_w + g * GK
            pltpu.sync_copy(d.at[pl.ds(crow, GK)], idx_v)
            cps = [pltpu.async_copy(ones_v, acc.at[idx_v.at[b]], sem, add=True)
                   for b in range(GK)]
            for cp in cps:
                cp.wait()
            return carry

        lax.fori_loop(0, ngrp, body, 0)
        plsc.subcore_barrier()
        pltpu.sync_copy(acc.at[pl.ds(sid * RPT, RPT)],
                        out.at[cid, pl.ds(sid * RPT, RPT)])

    return k


_EG = (EPAD // EB,)


def _k_cat_lin_stats(xi, xj, w1, b1):
    fo = w1.shape[1]

    def body(xi_r, xj_r, w1_r, b1_r, st_r):
        i = pl.program_id(0)
        xiv = xi_r[...].astype(F32)
        xjv = xj_r[...].astype(F32)
        y = _relu(_mm(_cat2(xiv, xjv), w1_r[...]) + b1_r[...])
        _acc_stats(i, y, st_r)

    return pl.pallas_call(
        body, grid=_EG,
        in_specs=[_ebs(xi.shape[1]), _ebs(xj.shape[1]), _fix(w1.shape),
                  _fix(b1.shape)],
        out_specs=_fix((2, fo)),
        out_shape=jax.ShapeDtypeStruct((2, fo), F32),
    )(xi, xj, w1, b1)


def _k_cat_lin_bn_lin_stats(xi, xj, w1, b1, ab, w2, b2):
    fo = w2.shape[1]

    def body(xi_r, xj_r, w1_r, b1_r, ab_r, w2_r, b2_r, st_r):
        i = pl.program_id(0)
        y1 = _relu(_mm(_cat2(xi_r[...], xj_r[...]), w1_r[...]) + b1_r[...])
        yh = y1 * ab_r[0:1, :] + ab_r[1:2, :]
        y2 = _relu(_mm(yh, w2_r[...]) + b2_r[...])
        _acc_stats(i, y2, st_r)

    return pl.pallas_call(
        body, grid=_EG,
        in_specs=[_ebs(xi.shape[1]), _ebs(xj.shape[1]), _fix(w1.shape),
                  _fix(b1.shape), _fix(ab.shape), _fix(w2.shape),
                  _fix(b2.shape)],
        out_specs=_fix((2, fo)),
        out_shape=jax.ShapeDtypeStruct((2, fo), F32),
    )(xi, xj, w1, b1, ab, w2, b2)


def _k_enc1_final(xi, xj, w1, b1, ab1, w2, b2, ab2, w3, b3):
    def body(xi_r, xj_r, w1_r, b1_r, ab1_r, w2_r, b2_r, ab2_r, w3_r, b3_r,
             o_r, st_r):
        i = pl.program_id(0)
        y1 = _relu(_mm(_cat2(xi_r[...], xj_r[...]), w1_r[...]) + b1_r[...])
        yh1 = y1 * ab1_r[0:1, :] + ab1_r[1:2, :]
        y2 = _relu(_mm(yh1, w2_r[...]) + b2_r[...])
        yh2 = y2 * ab2_r[0:1, :] + ab2_r[1:2, :]
        y3 = _relu(_mm(yh2, w3_r[...]) + b3_r[...])
        o_r[0] = y3
        _acc_stats(i, y3, st_r)

    return pl.pallas_call(
        body, grid=_EG,
        in_specs=[_ebs(xi.shape[1]), _ebs(xj.shape[1]), _fix(w1.shape),
                  _fix(b1.shape), _fix(ab1.shape), _fix(w2.shape),
                  _fix(b2.shape), _fix(ab2.shape), _fix(w3.shape),
                  _fix(b3.shape)],
        out_specs=[_ebs3(1, 32), _fix((2, 32))],
        out_shape=[jax.ShapeDtypeStruct((1, EPAD, 32), F32),
                   jax.ShapeDtypeStruct((2, 32), F32)],
    )(xi, xj, w1, b1, ab1, w2, b2, ab2, w3, b3)


def _k_cat_lin(xi, xj, w1, b1):
    fo = w1.shape[1]

    def body(xi_r, xj_r, w1_r, b1_r, y_r, st_r):
        i = pl.program_id(0)
        xiv = xi_r[...].astype(F32)
        xjv = xj_r[...].astype(F32)
        y = _relu(_mm(_cat2(xiv, xjv), w1_r[...]) + b1_r[...])
        y_r[...] = y
        _acc_stats(i, y, st_r)

    return pl.pallas_call(
        body, grid=_EG,
        in_specs=[_ebs(xi.shape[1]), _ebs(xj.shape[1]), _fix(w1.shape),
                  _fix(b1.shape)],
        out_specs=[_ebs(fo), _fix((2, fo))],
        out_shape=[jax.ShapeDtypeStruct((EPAD, fo), F32),
                   jax.ShapeDtypeStruct((2, fo), F32)],
    )(xi, xj, w1, b1)


def _k_bn_lin_raw(y, ab, w, b, halves):
    fo = w.shape[1]

    def body(y_r, ab_r, w_r, b_r, o_r, st_r):
        i = pl.program_id(0)
        yh = y_r[...] * ab_r[0:1, :] + ab_r[1:2, :]
        yo = _relu(_mm(yh, w_r[...]) + b_r[...])
        if halves:
            o_r[0] = yo[:, 0:fo // 2]
            o_r[1] = yo[:, fo // 2:fo]
        else:
            o_r[0] = yo
        _acc_stats(i, yo, st_r)

    c = 2 if halves else 1
    fh = fo // 2 if halves else fo
    return pl.pallas_call(
        body, grid=_EG,
        in_specs=[_ebs(y.shape[1]), _fix(ab.shape), _fix(w.shape), _fix(b.shape)],
        out_specs=[_ebs3(c, fh), _fix((2, fo))],
        out_shape=[jax.ShapeDtypeStruct((c, EPAD, fh), F32),
                   jax.ShapeDtypeStruct((2, fo), F32)],
    )(y, ab, w, b)


def _k_bn_lin_plain(y, ab, w, b):
    fo = w.shape[1]

    def body(y_r, ab_r, w_r, b_r, o_r, st_r):
        i = pl.program_id(0)
        yh = y_r[...] * ab_r[0:1, :] + ab_r[1:2, :]
        yo = _relu(_mm(yh, w_r[...]) + b_r[...])
        o_r[...] = yo
        _acc_stats(i, yo, st_r)

    return pl.pallas_call(
        body, grid=_EG,
        in_specs=[_ebs(y.shape[1]), _fix(ab.shape), _fix(w.shape), _fix(b.shape)],
        out_specs=[_ebs(fo), _fix((2, fo))],
        out_shape=[jax.ShapeDtypeStruct((EPAD, fo), F32),
                   jax.ShapeDtypeStruct((2, fo), F32)],
    )(y, ab, w, b)


def _k_d1_stats(xi, xj, w1, b1, w2, b2):
    def body(xi_r, xj_r, w1_r, b1_r, w2_r, b2_r, st_r):
        i = pl.program_id(0)
        zi = xi_r[...][:, 0:2]
        zj = xj_r[...][:, 0:2]
        y1 = _relu(_mm(_cat2(zi, zj), w1_r[...]) + b1_r[...])
        t2 = _mm(y1, w2_r[...]) + b2_r[...]
        _acc_stats(i, t2, st_r)

    return pl.pallas_call(
        body, grid=_EG,
        in_specs=[_ebs(16), _ebs(16), _fix(w1.shape), _fix(b1.shape),
                  _fix(w2.shape), _fix(b2.shape)],
        out_specs=_fix((2, 32)),
        out_shape=jax.ShapeDtypeStruct((2, 32), F32),
    )(xi, xj, w1, b1, w2, b2)


def _k_d1_main(xi, xj, w1, b1, w2, b2, ab2, w3, b3):
    def body(xi_r, xj_r, w1_r, b1_r, w2_r, b2_r, ab_r, w3_r, b3_r, o_r, st_r):
        i = pl.program_id(0)
        zi = xi_r[...][:, 0:2]
        zj = xj_r[...][:, 0:2]
        y1 = _relu(_mm(_cat2(zi, zj), w1_r[...]) + b1_r[...])
        t2 = _mm(y1, w2_r[...]) + b2_r[...]
        y2 = _relu(t2 * ab_r[0:1, :] + ab_r[1:2, :])
        y3 = _relu(_mm(y2, w3_r[...]) + b3_r[...])
        o_r[0] = y3[:, 0:32]
        o_r[1] = y3[:, 32:64]
        _acc_stats(i, y3, st_r)

    return pl.pallas_call(
        body, grid=_EG,
        in_specs=[_ebs(16), _ebs(16), _fix(w1.shape), _fix(b1.shape),
                  _fix(w2.shape), _fix(b2.shape), _fix(ab2.shape),
                  _fix(w3.shape), _fix(b3.shape)],
        out_specs=[_ebs3(2, 32), _fix((2, 64))],
        out_shape=[jax.ShapeDtypeStruct((2, EPAD, 32), F32),
                   jax.ShapeDtypeStruct((2, 64), F32)],
    )(xi, xj, w1, b1, w2, b2, ab2, w3, b3)


def _k_final_edge(y2, ab2, w3, b3):
    def body(y_r, ab_r, w_r, b_r, o_r):
        yh = y_r[...] * ab_r[0:1, :] + ab_r[1:2, :]
        o_r[0] = _mm(yh, w_r[...]) + b_r[...]

    return pl.pallas_call(
        body, grid=_EG,
        in_specs=[_ebs(y2.shape[1]), _fix(ab2.shape), _fix(w3.shape),
                  _fix(b3.shape)],
        out_specs=_ebs3(1, 16),
        out_shape=jax.ShapeDtypeStruct((1, EPAD, 16), F32),
    )(y2, ab2, w3, b3)


_NG = (NN // NBK,)


def _k_xstats(x):
    def body(x_r, st_r):
        i = pl.program_id(0)
        xv = x_r[...]
        blk = jnp.concatenate(
            [jnp.sum(xv, 0, keepdims=True), jnp.sum(xv * xv, 0, keepdims=True)], 0)

        @pl.when(i == 0)
        def _():
            st_r[...] = jnp.zeros_like(st_r)

        st_r[...] += blk

    return pl.pallas_call(
        body, grid=_NG, in_specs=[_nbs(16)], out_specs=_fix((2, 16)),
        out_shape=jax.ShapeDtypeStruct((2, 16), F32))(x)


def _k_bn0(x, st, g, b):
    def body(x_r, st_r, g_r, b_r, o_r):
        m = st_r[0:1, :] / float(NN)
        v = st_r[1:2, :] / float(NN) - m * m
        o_r[...] = g_r[...] * (x_r[...] - m) / jnp.sqrt(v + 1e-5) + b_r[...]

    return pl.pallas_call(
        body, grid=_NG,
        in_specs=[_nbs(16), _fix((2, 16)), _fix((1, 16)), _fix((1, 16))],
        out_specs=_nbs(16),
        out_shape=jax.ShapeDtypeStruct((NN, 16), F32))(x, st, g, b)


def _k_node1(s0, s1, c0, c1, ab):
    f = s0.shape[1]

    def body(s0_r, s1_r, c0_r, c1_r, ab_r, h_r, rec_r, pos_r):
        cnt = c0_r[...] + c1_r[...]
        rec = 1.0 / jnp.maximum(cnt, 1.0)
        pos = jnp.where(cnt > 0.0, 1.0, 0.0)
        mean = (s0_r[...] + s1_r[...]) * rec
        h_r[...] = (mean * ab_r[0:1, :] + ab_r[1:2, :]) * pos
        rec_r[...] = rec
        pos_r[...] = pos

    return pl.pallas_call(
        body, grid=_NG,
        in_specs=[_nbs(f), _nbs(f), _nbs(1), _nbs(1), _fix(ab.shape)],
        out_specs=[_nbs(f), _nbs(1), _nbs(1)],
        out_shape=[jax.ShapeDtypeStruct((NN, f), F32),
                   jax.ShapeDtypeStruct((NN, 1), F32),
                   jax.ShapeDtypeStruct((NN, 1), F32)],
    )(s0, s1, c0, c1, ab)


def _k_node_mid(s0, s1, rec, pos, ab, eps, hw):
    (wmu, bmu, wvar, bvar, wd, bd, wd1, bd1, wd2, bd2, wbf, bbf) = hw

    def body(s0_r, s1_r, rec_r, pos_r, ab_r, eps_r, wmu_r, bmu_r, wvar_r,
             bvar_r, wd_r, bd_r, wd1_r, bd1_r, wd2_r, bd2_r, wbf_r, bbf_r,
             mu_r, lv_r, z0_r, zk_r, ldj_r, zp_r):
        mean = (s0_r[...] + s1_r[...]) * rec_r[...]
        h = (mean * ab_r[0:1, :] + ab_r[1:2, :]) * pos_r[...]
        mu = _mm(h, wmu_r[...]) + bmu_r[...]
        lv = _mm(h, wvar_r[...]) + bvar_r[...]
        fd = _mm(h, wd_r[...]) + bd_r[...]
        d1 = jnp.tanh(_mm(h, wd1_r[...]) + bd1_r[...])
        d2 = jnp.tanh(_mm(h, wd2_r[...]) + bd2_r[...])
        bf = _mm(h, wbf_r[...]) + bbf_r[...]
        z0 = mu + eps_r[...] * jnp.exp(0.5 * lv)
        zc0 = z0[:, 0:1]
        zc1 = z0[:, 1:2]
        ldj = jnp.zeros_like(zc0)
        for k in range(6):
            fd01 = fd[:, 6 + k:7 + k]
            fd10 = fd[:, 12 + k:13 + k]
            d1_0 = d1[:, k:k + 1]
            d1_1 = d1[:, 6 + k:7 + k]
            d2_0 = d2[:, k:k + 1]
            d2_1 = d2[:, 6 + k:7 + k]
            b_0 = bf[:, k:k + 1]
            b_1 = bf[:, 6 + k:7 + k]
            if k % 2 == 1:
                zp0, zp1 = zc1, zc0
            else:
                zp0, zp1 = zc0, zc1
            t0 = jnp.tanh(zp0 * d2_0 + zp1 * fd10 + b_0)
            t1 = jnp.tanh(zp1 * d2_1 + b_1)
            n0 = t0 * d1_0 + t1 * fd01
            n1 = t1 * d1_1
            if k % 2 == 1:
                n0, n1 = n1, n0
            zc0 = zc0 + n0
            zc1 = zc1 + n1
            dj0 = (1.0 - t0 * t0) * d1_0 * d2_0 + 1.0
            dj1 = (1.0 - t1 * t1) * d1_1 * d2_1 + 1.0
            ldj = ldj + jnp.log(jnp.abs(dj0)) + jnp.log(jnp.abs(dj1))
        zk = jnp.concatenate([zc0, zc1], axis=1)
        mu_r[...] = mu
        lv_r[...] = lv
        z0_r[...] = z0
        zk_r[...] = zk
        ldj_r[...] = ldj
        zp_r[...] = jnp.concatenate(
            [zk, jnp.zeros((zk.shape[0], 14), F32)], axis=1)

    small = [wmu, bmu, wvar, bvar, wd, bd, wd1, bd1, wd2, bd2, wbf, bbf]
    return pl.pallas_call(
        body, grid=_NG,
        in_specs=[_nbs(32), _nbs(32), _nbs(1), _nbs(1), _fix(ab.shape),
                  _nbs(2)] + [_fix(a.shape) for a in small],
        out_specs=[_nbs(2), _nbs(2), _nbs(2), _nbs(2), _nbs(1), _nbs(16)],
        out_shape=[jax.ShapeDtypeStruct((NN, 2), F32),
                   jax.ShapeDtypeStruct((NN, 2), F32),
                   jax.ShapeDtypeStruct((NN, 2), F32),
                   jax.ShapeDtypeStruct((NN, 2), F32),
                   jax.ShapeDtypeStruct((NN, 1), F32),
                   jax.ShapeDtypeStruct((NN, 16), F32)],
    )(s0, s1, rec, pos, ab, eps, *small)


def _k_node_halves(sa, sb, rec, pos, ab):
    def body(sa_r, sb_r, rec_r, pos_r, ab_r, h_r):
        mean = jnp.concatenate([sa_r[...], sb_r[...]], axis=1) * rec_r[...]
        h = (mean * ab_r[0:1, :] + ab_r[1:2, :]) * pos_r[...]
        h_r[...] = h.astype(jnp.bfloat16)

    return pl.pallas_call(
        body, grid=_NG,
        in_specs=[_nbs(32), _nbs(32), _nbs(1), _nbs(1), _fix(ab.shape)],
        out_specs=_nbs(64),
        out_shape=jax.ShapeDtypeStruct((NN, 64), jnp.bfloat16),
    )(sa, sb, rec, pos, ab)


def _k_node_final(s0, s1, rec):
    def body(s0_r, s1_r, rec_r, o_r):
        o_r[...] = (s0_r[...] + s1_r[...]) * rec_r[...]

    return pl.pallas_call(
        body, grid=_NG,
        in_specs=[_nbs(16), _nbs(16), _nbs(1)],
        out_specs=_nbs(16),
        out_shape=jax.ShapeDtypeStruct((NN, 16), F32),
    )(s0, s1, rec)


def kernel(x, edge_index, eps, params):
    p = params
    src = edge_index[0]
    dst = edge_index[1]
    padlen = EPAD - EE
    zpad_i = jnp.zeros((padlen,), jnp.int32)
    dst_g = jnp.concatenate([dst, zpad_i]).reshape(EPAD // CH, CH)
    src_g = jnp.concatenate([src, zpad_i]).reshape(EPAD // CH, CH)
    dst_s = jnp.concatenate(
        [dst, jnp.full((padlen,), NN, jnp.int32)]).reshape(EPAD // CH, CH)
    z32 = jnp.zeros((RPT, 32), F32)
    z16 = jnp.zeros((RPT, 16), F32)
    ones16 = jnp.ones((CH, 16), F32)

    cntp = _counts()(dst_s, z16, ones16)
    c0 = cntp[0, :NN, 0:1]
    c1 = cntp[1, :NN, 0:1]

    stx = _k_xstats(x)
    xb = _k_bn0(x, stx, p["bn0"]["g"][None, :], p["bn0"]["b"][None, :])
    xi, xj = _gather2(16, 16)(xb, dst_g, xb, src_g)
    w1 = p["enc1"]["l1"]["W"]
    b1 = p["enc1"]["l1"]["b"][None, :]
    st1 = _k_cat_lin_stats(xi, xj, w1, b1)
    ab1 = _ab(st1, p["enc1"]["bn1"]["g"], p["enc1"]["bn1"]["b"], float(EE))
    st2 = _k_cat_lin_bn_lin_stats(xi, xj, w1, b1, ab1,
                                  p["enc1"]["l2"]["W"],
                                  p["enc1"]["l2"]["b"][None, :])
    ab2 = _ab(st2, p["enc1"]["bn2"]["g"], p["enc1"]["bn2"]["b"], float(EE))
    y3, st3 = _k_enc1_final(xi, xj, w1, b1, ab1, p["enc1"]["l2"]["W"],
                            p["enc1"]["l2"]["b"][None, :], ab2,
                            p["enc1"]["l3"]["W"], p["enc1"]["l3"]["b"][None, :])
    ab3 = _ab(st3, p["enc1"]["bn3"]["g"], p["enc1"]["bn3"]["b"], float(EE))
    s1p = _scatter(False, 32)(y3, dst_s, z32)
    h1, rec, pos = _k_node1(s1p[0, :NN], s1p[1, :NN], c0, c1, ab3)

    g2a, g2b = _gather2(32, 32)(h1, dst_g, h1, src_g)
    y1e, st1e = _k_cat_lin(g2a, g2b, p["enc2"]["l1"]["W"],
                           p["enc2"]["l1"]["b"][None, :])
    ab1e = _ab(st1e, p["enc2"]["bn1"]["g"], p["enc2"]["bn1"]["b"], float(EE))
    y2e, st2e = _k_bn_lin_raw(y1e, ab1e, p["enc2"]["l2"]["W"],
                              p["enc2"]["l2"]["b"][None, :], halves=False)
    ab2e = _ab(st2e, p["enc2"]["bn2"]["g"], p["enc2"]["bn2"]["b"], float(EE))
    s2p = _scatter(False, 32)(y2e, dst_s, z32)

    hw = (p["mu"]["W"], p["mu"]["b"][None, :],
          p["var"]["W"], p["var"]["b"][None, :],
          p["amor_d"]["W"], p["amor_d"]["b"][None, :],
          p["amor_diag1"]["W"], p["amor_diag1"]["b"][None, :],
          p["amor_diag2"]["W"], p["amor_diag2"]["b"][None, :],
          p["amor_b"]["W"], p["amor_b"]["b"][None, :])
    mu, log_var, z0, zk, ldj, zkpad = _k_node_mid(
        s2p[0, :NN], s2p[1, :NN], rec, pos, ab2e, eps, hw)

    g3a, g3b = _gather2(16, 16)(zkpad, dst_g, zkpad, src_g)
    w1d = p["dec1"]["l1"]["W"]
    b1d = p["dec1"]["l1"]["b"][None, :]
    w2d = p["dec1"]["l2"]["W"]
    b2d = p["dec1"]["l2"]["b"][None, :]
    st_t2 = _k_d1_stats(g3a, g3b, w1d, b1d, w2d, b2d)
    ab2d = _ab(st_t2, p["dec1"]["bn2"]["g"], p["dec1"]["bn2"]["b"], float(EE))
    y3h, st3d = _k_d1_main(g3a, g3b, w1d, b1d, w2d, b2d, ab2d,
                           p["dec1"]["l3"]["W"], p["dec1"]["l3"]["b"][None, :])
    ab3d = _ab(st3d, p["dec1"]["bn3"]["g"], p["dec1"]["bn3"]["b"], float(EE))
    s3p = _scatter(True, 32)(y3h, dst_s, z32)
    h2 = _k_node_halves(s3p[0, :NN], s3p[1, :NN], rec, pos, ab3d)

    g4a, g4b = _gather2(64, 64, jnp.bfloat16)(h2, dst_g, h2, src_g)
    y1f, st1f = _k_cat_lin(g4a, g4b, p["dec2"]["l1"]["W"],
                           p["dec2"]["l1"]["b"][None, :])
    ab1f = _ab(st1f, p["dec2"]["bn1"]["g"], p["dec2"]["bn1"]["b"], float(EE))
    y2f, st2f = _k_bn_lin_plain(y1f, ab1f, p["dec2"]["l2"]["W"],
                                p["dec2"]["l2"]["b"][None, :])
    ab2f = _ab(st2f, p["dec2"]["bn2"]["g"], p["dec2"]["bn2"]["b"], float(EE))
    m4 = _k_final_edge(y2f, ab2f, p["dec2"]["l3"]["W"],
                       p["dec2"]["l3"]["b"][None, :])
    s4p = _scatter(False, 16)(m4, dst_s, z16)
    x_decoded = _k_node_final(s4p[0, :NN], s4p[1, :NN], rec)

    return (x_decoded, mu, log_var, ldj[:, 0], z0, zk)


# speedup vs baseline: 1.0106x; 1.0106x over previous
import functools

import jax
import jax.numpy as jnp
from jax import lax
from jax.experimental import pallas as pl
from jax.experimental.pallas import tpu as pltpu
from jax.experimental.pallas import tpu_sc as plsc

NN = 50000
EE = 800000
EPAD = 802816
NPAD = 50016
EB = 8192
NBK = 2000
CH = 128
RPT = NPAD // 16
F32 = jnp.float32


def _fix(shape):
    return pl.BlockSpec(shape, lambda i: (0,) * len(shape))


def _ebs(f):
    return pl.BlockSpec((EB, f), lambda i: (i, 0))


def _ebs3(c, f):
    return pl.BlockSpec((c, EB, f), lambda i: (0, i, 0))


def _nbs(f):
    return pl.BlockSpec((NBK, f), lambda i: (i, 0))


def _relu(v):
    return jnp.maximum(v, 0.0)


def _mm(a, b):
    return jnp.dot(a, b, preferred_element_type=F32)


def _cat2(xi, xj):
    return jnp.concatenate([xi, xj - xi], axis=1)


def _acc_stats(i, y, st_ref):
    mask = (i * EB + lax.broadcasted_iota(jnp.int32, (EB, 1), 0)) < EE
    ym = jnp.where(mask, y, 0.0)
    blk = jnp.concatenate(
        [jnp.sum(ym, 0, keepdims=True), jnp.sum(ym * ym, 0, keepdims=True)], 0)

    @pl.when(i == 0)
    def _():
        st_ref[...] = jnp.zeros_like(st_ref)

    st_ref[...] += blk


def _ab(st, g, b, n):
    m = st[0] / n
    v = st[1] / n - m * m
    a = g / jnp.sqrt(v + 1e-5)
    return jnp.stack([a, b - m * a])


def _sc_mesh():
    return plsc.VectorSubcoreMesh(core_axis_name="c", subcore_axis_name="s")


_SC_PARAMS = pltpu.CompilerParams(use_tc_tiling_on_sc=False)


GK = 14
GKS = 7


@functools.lru_cache(maxsize=None)
def _gather2(fa, fb, dt=jnp.float32):
    rows_w = (EPAD // CH) // 32
    ngrp = rows_w // GK

    @functools.partial(
        pl.kernel, mesh=_sc_mesh(), compiler_params=_SC_PARAMS,
        out_type=[jax.ShapeDtypeStruct((EPAD, fa), dt),
                  jax.ShapeDtypeStruct((EPAD, fb), dt)],
        scratch_types=[pltpu.VMEM((GK, CH), jnp.int32),
                       pltpu.VMEM((GK * CH, fa), dt),
                       pltpu.VMEM((GK, CH), jnp.int32),
                       pltpu.VMEM((GK * CH, fb), dt),
                       pltpu.SemaphoreType.DMA,
                       pltpu.SemaphoreType.DMA],
    )
    def k(ta, ia, tb, ib, outa, outb, iva, rva, ivb, rvb, sa, sb):
        wid = lax.axis_index("c") * 16 + lax.axis_index("s")

        def body(g, carry):
            crow = wid * rows_w + g * GK
            base = crow * CH
            pltpu.sync_copy(ia.at[pl.ds(crow, GK)], iva)
            pltpu.sync_copy(ib.at[pl.ds(crow, GK)], ivb)
            cps = []
            for b in range(GK):
                cps.append(pltpu.async_copy(
                    ta.at[iva.at[b]], rva.at[pl.ds(b * CH, CH)], sa))
                cps.append(pltpu.async_copy(
                    tb.at[ivb.at[b]], rvb.at[pl.ds(b * CH, CH)], sb))
            for cp in cps:
                cp.wait()
            pltpu.sync_copy(rva, outa.at[pl.ds(base, GK * CH)])
            pltpu.sync_copy(rvb, outb.at[pl.ds(base, GK * CH)])
            return carry

        lax.fori_loop(0, ngrp, body, 0)

    return k


@functools.lru_cache(maxsize=None)
def _scatter(feature_split, fw):
    if feature_split:
        rows_w = (EPAD // CH) // 16
    else:
        rows_w = (EPAD // CH) // 32
    ngrp = rows_w // GKS

    @functools.partial(
        pl.kernel, mesh=_sc_mesh(), compiler_params=_SC_PARAMS,
        out_type=jax.ShapeDtypeStruct((2, NPAD, fw), F32),
        scratch_types=[pltpu.VMEM((GKS, CH), jnp.int32),
                       pltpu.VMEM((GKS * CH, fw), F32),
                       pltpu.VMEM_SHARED((NPAD, fw), F32),
                       pltpu.SemaphoreType.DMA],
    )
    def k(y, d, z, out, idx_v, row_v, acc, sem):
        cid = lax.axis_index("c")
        sid = lax.axis_index("s")
        pltpu.sync_copy(z, acc.at[pl.ds(sid * RPT, RPT)])
        plsc.subcore_barrier()

        def body(g, carry):
            if feature_split:
                crow = sid * rows_w + g * GKS
                pltpu.sync_copy(y.at[cid, pl.ds(crow * CH, GKS * CH)], row_v)
            else:
                crow = (cid * 16 + sid) * rows_w + g * GKS
                pltpu.sync_copy(y.at[0, pl.ds(crow * CH, GKS * CH)], row_v)
            pltpu.sync_copy(d.at[pl.ds(crow, GKS)], idx_v)
            cps = [pltpu.async_copy(row_v.at[pl.ds(b * CH, CH)],
                                    acc.at[idx_v.at[b]], sem, add=True)
                   for b in range(GKS)]
            for cp in cps:
                cp.wait()
            return carry

        lax.fori_loop(0, ngrp, body, 0)
        plsc.subcore_barrier()
        pltpu.sync_copy(acc.at[pl.ds(sid * RPT, RPT)],
                        out.at[cid, pl.ds(sid * RPT, RPT)])

    return k


@functools.lru_cache(maxsize=None)
def _counts():
    rows_w = (EPAD // CH) // 32
    ngrp = rows_w // GKS

    @functools.partial(
        pl.kernel, mesh=_sc_mesh(), compiler_params=_SC_PARAMS,
        out_type=jax.ShapeDtypeStruct((2, NPAD, 16), F32),
        scratch_types=[pltpu.VMEM((GKS, CH), jnp.int32),
                       pltpu.VMEM((CH, 16), F32),
                       pltpu.VMEM_SHARED((NPAD, 16), F32),
                       pltpu.SemaphoreType.DMA],
    )
    def k(d, z, ones_hbm, out, idx_v, ones_v, acc, sem):
        cid = lax.axis_index("c")
        sid = lax.axis_index("s")
        pltpu.sync_copy(z, acc.at[pl.ds(sid * RPT, RPT)])
        pltpu.sync_copy(ones_hbm, ones_v)
        plsc.subcore_barrier()

        def body(g, carry):
            crow = (cid * 16 + sid) * rows_w + g * GKS
            pltpu.sync_copy(d.at[pl.ds(crow, GKS)], idx_v)
            cps = [pltpu.async_copy(ones_v, acc.at[idx_v.at[b]], sem, add=True)
                   for b in range(GKS)]
            for cp in cps:
                cp.wait()
            return carry

        lax.fori_loop(0, ngrp, body, 0)
        plsc.subcore_barrier()
        pltpu.sync_copy(acc.at[pl.ds(sid * RPT, RPT)],
                        out.at[cid, pl.ds(sid * RPT, RPT)])

    return k


_EG = (EPAD // EB,)


def _k_cat_lin_stats(xi, xj, w1, b1):
    fo = w1.shape[1]

    def body(xi_r, xj_r, w1_r, b1_r, st_r):
        i = pl.program_id(0)
        xiv = xi_r[...].astype(F32)
        xjv = xj_r[...].astype(F32)
        y = _relu(_mm(_cat2(xiv, xjv), w1_r[...]) + b1_r[...])
        _acc_stats(i, y, st_r)

    return pl.pallas_call(
        body, grid=_EG,
        in_specs=[_ebs(xi.shape[1]), _ebs(xj.shape[1]), _fix(w1.shape),
                  _fix(b1.shape)],
        out_specs=_fix((2, fo)),
        out_shape=jax.ShapeDtypeStruct((2, fo), F32),
    )(xi, xj, w1, b1)


def _k_cat_lin_bn_lin(xi, xj, w1, b1, ab, w2, b2):
    fo = w2.shape[1]

    def body(xi_r, xj_r, w1_r, b1_r, ab_r, w2_r, b2_r, y2_r, st_r):
        i = pl.program_id(0)
        y1 = _relu(_mm(_cat2(xi_r[...], xj_r[...]), w1_r[...]) + b1_r[...])
        yh = y1 * ab_r[0:1, :] + ab_r[1:2, :]
        y2 = _relu(_mm(yh, w2_r[...]) + b2_r[...])
        y2_r[...] = y2
        _acc_stats(i, y2, st_r)

    return pl.pallas_call(
        body, grid=_EG,
        in_specs=[_ebs(xi.shape[1]), _ebs(xj.shape[1]), _fix(w1.shape),
                  _fix(b1.shape), _fix(ab.shape), _fix(w2.shape),
                  _fix(b2.shape)],
        out_specs=[_ebs(fo), _fix((2, fo))],
        out_shape=[jax.ShapeDtypeStruct((EPAD, fo), F32),
                   jax.ShapeDtypeStruct((2, fo), F32)],
    )(xi, xj, w1, b1, ab, w2, b2)


def _k_cat_lin(xi, xj, w1, b1):
    fo = w1.shape[1]

    def body(xi_r, xj_r, w1_r, b1_r, y_r, st_r):
        i = pl.program_id(0)
        xiv = xi_r[...].astype(F32)
        xjv = xj_r[...].astype(F32)
        y = _relu(_mm(_cat2(xiv, xjv), w1_r[...]) + b1_r[...])
        y_r[...] = y
        _acc_stats(i, y, st_r)

    return pl.pallas_call(
        body, grid=_EG,
        in_specs=[_ebs(xi.shape[1]), _ebs(xj.shape[1]), _fix(w1.shape),
                  _fix(b1.shape)],
        out_specs=[_ebs(fo), _fix((2, fo))],
        out_shape=[jax.ShapeDtypeStruct((EPAD, fo), F32),
                   jax.ShapeDtypeStruct((2, fo), F32)],
    )(xi, xj, w1, b1)


def _k_bn_lin_raw(y, ab, w, b, halves):
    fo = w.shape[1]

    def body(y_r, ab_r, w_r, b_r, o_r, st_r):
        i = pl.program_id(0)
        yh = y_r[...] * ab_r[0:1, :] + ab_r[1:2, :]
        yo = _relu(_mm(yh, w_r[...]) + b_r[...])
        if halves:
            o_r[0] = yo[:, 0:fo // 2]
            o_r[1] = yo[:, fo // 2:fo]
        else:
            o_r[0] = yo
        _acc_stats(i, yo, st_r)

    c = 2 if halves else 1
    fh = fo // 2 if halves else fo
    return pl.pallas_call(
        body, grid=_EG,
        in_specs=[_ebs(y.shape[1]), _fix(ab.shape), _fix(w.shape), _fix(b.shape)],
        out_specs=[_ebs3(c, fh), _fix((2, fo))],
        out_shape=[jax.ShapeDtypeStruct((c, EPAD, fh), F32),
                   jax.ShapeDtypeStruct((2, fo), F32)],
    )(y, ab, w, b)


def _k_bn_lin_plain(y, ab, w, b):
    fo = w.shape[1]

    def body(y_r, ab_r, w_r, b_r, o_r, st_r):
        i = pl.program_id(0)
        yh = y_r[...] * ab_r[0:1, :] + ab_r[1:2, :]
        yo = _relu(_mm(yh, w_r[...]) + b_r[...])
        o_r[...] = yo
        _acc_stats(i, yo, st_r)

    return pl.pallas_call(
        body, grid=_EG,
        in_specs=[_ebs(y.shape[1]), _fix(ab.shape), _fix(w.shape), _fix(b.shape)],
        out_specs=[_ebs(fo), _fix((2, fo))],
        out_shape=[jax.ShapeDtypeStruct((EPAD, fo), F32),
                   jax.ShapeDtypeStruct((2, fo), F32)],
    )(y, ab, w, b)


def _k_d1_stats(xi, xj, w1, b1, w2, b2):
    def body(xi_r, xj_r, w1_r, b1_r, w2_r, b2_r, st_r):
        i = pl.program_id(0)
        zi = xi_r[...][:, 0:2]
        zj = xj_r[...][:, 0:2]
        y1 = _relu(_mm(_cat2(zi, zj), w1_r[...]) + b1_r[...])
        t2 = _mm(y1, w2_r[...]) + b2_r[...]
        _acc_stats(i, t2, st_r)

    return pl.pallas_call(
        body, grid=_EG,
        in_specs=[_ebs(16), _ebs(16), _fix(w1.shape), _fix(b1.shape),
                  _fix(w2.shape), _fix(b2.shape)],
        out_specs=_fix((2, 32)),
        out_shape=jax.ShapeDtypeStruct((2, 32), F32),
    )(xi, xj, w1, b1, w2, b2)


def _k_d1_main(xi, xj, w1, b1, w2, b2, ab2, w3, b3):
    def body(xi_r, xj_r, w1_r, b1_r, w2_r, b2_r, ab_r, w3_r, b3_r, o_r, st_r):
        i = pl.program_id(0)
        zi = xi_r[...][:, 0:2]
        zj = xj_r[...][:, 0:2]
        y1 = _relu(_mm(_cat2(zi, zj), w1_r[...]) + b1_r[...])
        t2 = _mm(y1, w2_r[...]) + b2_r[...]
        y2 = _relu(t2 * ab_r[0:1, :] + ab_r[1:2, :])
        y3 = _relu(_mm(y2, w3_r[...]) + b3_r[...])
        o_r[0] = y3[:, 0:32]
        o_r[1] = y3[:, 32:64]
        _acc_stats(i, y3, st_r)

    return pl.pallas_call(
        body, grid=_EG,
        in_specs=[_ebs(16), _ebs(16), _fix(w1.shape), _fix(b1.shape),
                  _fix(w2.shape), _fix(b2.shape), _fix(ab2.shape),
                  _fix(w3.shape), _fix(b3.shape)],
        out_specs=[_ebs3(2, 32), _fix((2, 64))],
        out_shape=[jax.ShapeDtypeStruct((2, EPAD, 32), F32),
                   jax.ShapeDtypeStruct((2, 64), F32)],
    )(xi, xj, w1, b1, w2, b2, ab2, w3, b3)


def _k_final_edge(y2, ab2, w3, b3):
    def body(y_r, ab_r, w_r, b_r, o_r):
        yh = y_r[...] * ab_r[0:1, :] + ab_r[1:2, :]
        o_r[0] = _mm(yh, w_r[...]) + b_r[...]

    return pl.pallas_call(
        body, grid=_EG,
        in_specs=[_ebs(y2.shape[1]), _fix(ab2.shape), _fix(w3.shape),
                  _fix(b3.shape)],
        out_specs=_ebs3(1, 16),
        out_shape=jax.ShapeDtypeStruct((1, EPAD, 16), F32),
    )(y2, ab2, w3, b3)


_NG = (NN // NBK,)


def _k_xstats(x):
    def body(x_r, st_r):
        i = pl.program_id(0)
        xv = x_r[...]
        blk = jnp.concatenate(
            [jnp.sum(xv, 0, keepdims=True), jnp.sum(xv * xv, 0, keepdims=True)], 0)

        @pl.when(i == 0)
        def _():
            st_r[...] = jnp.zeros_like(st_r)

        st_r[...] += blk

    return pl.pallas_call(
        body, grid=_NG, in_specs=[_nbs(16)], out_specs=_fix((2, 16)),
        out_shape=jax.ShapeDtypeStruct((2, 16), F32))(x)


def _k_bn0(x, st, g, b):
    def body(x_r, st_r, g_r, b_r, o_r):
        m = st_r[0:1, :] / float(NN)
        v = st_r[1:2, :] / float(NN) - m * m
        o_r[...] = g_r[...] * (x_r[...] - m) / jnp.sqrt(v + 1e-5) + b_r[...]

    return pl.pallas_call(
        body, grid=_NG,
        in_specs=[_nbs(16), _fix((2, 16)), _fix((1, 16)), _fix((1, 16))],
        out_specs=_nbs(16),
        out_shape=jax.ShapeDtypeStruct((NN, 16), F32))(x, st, g, b)


def _k_node1(s0, s1, c0, c1, ab):
    f = s0.shape[1]

    def body(s0_r, s1_r, c0_r, c1_r, ab_r, h_r, rec_r, pos_r):
        cnt = c0_r[...] + c1_r[...]
        rec = 1.0 / jnp.maximum(cnt, 1.0)
        pos = jnp.where(cnt > 0.0, 1.0, 0.0)
        mean = (s0_r[...] + s1_r[...]) * rec
        h_r[...] = (mean * ab_r[0:1, :] + ab_r[1:2, :]) * pos
        rec_r[...] = rec
        pos_r[...] = pos

    return pl.pallas_call(
        body, grid=_NG,
        in_specs=[_nbs(f), _nbs(f), _nbs(1), _nbs(1), _fix(ab.shape)],
        out_specs=[_nbs(f), _nbs(1), _nbs(1)],
        out_shape=[jax.ShapeDtypeStruct((NN, f), F32),
                   jax.ShapeDtypeStruct((NN, 1), F32),
                   jax.ShapeDtypeStruct((NN, 1), F32)],
    )(s0, s1, c0, c1, ab)


def _k_node_mid(s0, s1, rec, pos, ab, eps, hw):
    (wmu, bmu, wvar, bvar, wd, bd, wd1, bd1, wd2, bd2, wbf, bbf) = hw

    def body(s0_r, s1_r, rec_r, pos_r, ab_r, eps_r, wmu_r, bmu_r, wvar_r,
             bvar_r, wd_r, bd_r, wd1_r, bd1_r, wd2_r, bd2_r, wbf_r, bbf_r,
             mu_r, lv_r, z0_r, zk_r, ldj_r, zp_r):
        mean = (s0_r[...] + s1_r[...]) * rec_r[...]
        h = (mean * ab_r[0:1, :] + ab_r[1:2, :]) * pos_r[...]
        mu = _mm(h, wmu_r[...]) + bmu_r[...]
        lv = _mm(h, wvar_r[...]) + bvar_r[...]
        fd = _mm(h, wd_r[...]) + bd_r[...]
        d1 = jnp.tanh(_mm(h, wd1_r[...]) + bd1_r[...])
        d2 = jnp.tanh(_mm(h, wd2_r[...]) + bd2_r[...])
        bf = _mm(h, wbf_r[...]) + bbf_r[...]
        z0 = mu + eps_r[...] * jnp.exp(0.5 * lv)
        zc0 = z0[:, 0:1]
        zc1 = z0[:, 1:2]
        ldj = jnp.zeros_like(zc0)
        for k in range(6):
            fd01 = fd[:, 6 + k:7 + k]
            fd10 = fd[:, 12 + k:13 + k]
            d1_0 = d1[:, k:k + 1]
            d1_1 = d1[:, 6 + k:7 + k]
            d2_0 = d2[:, k:k + 1]
            d2_1 = d2[:, 6 + k:7 + k]
            b_0 = bf[:, k:k + 1]
            b_1 = bf[:, 6 + k:7 + k]
            if k % 2 == 1:
                zp0, zp1 = zc1, zc0
            else:
                zp0, zp1 = zc0, zc1
            t0 = jnp.tanh(zp0 * d2_0 + zp1 * fd10 + b_0)
            t1 = jnp.tanh(zp1 * d2_1 + b_1)
            n0 = t0 * d1_0 + t1 * fd01
            n1 = t1 * d1_1
            if k % 2 == 1:
                n0, n1 = n1, n0
            zc0 = zc0 + n0
            zc1 = zc1 + n1
            dj0 = (1.0 - t0 * t0) * d1_0 * d2_0 + 1.0
            dj1 = (1.0 - t1 * t1) * d1_1 * d2_1 + 1.0
            ldj = ldj + jnp.log(jnp.abs(dj0)) + jnp.log(jnp.abs(dj1))
        zk = jnp.concatenate([zc0, zc1], axis=1)
        mu_r[...] = mu
        lv_r[...] = lv
        z0_r[...] = z0
        zk_r[...] = zk
        ldj_r[...] = ldj
        zp_r[...] = jnp.concatenate(
            [zk, jnp.zeros((zk.shape[0], 14), F32)], axis=1)

    small = [wmu, bmu, wvar, bvar, wd, bd, wd1, bd1, wd2, bd2, wbf, bbf]
    return pl.pallas_call(
        body, grid=_NG,
        in_specs=[_nbs(32), _nbs(32), _nbs(1), _nbs(1), _fix(ab.shape),
                  _nbs(2)] + [_fix(a.shape) for a in small],
        out_specs=[_nbs(2), _nbs(2), _nbs(2), _nbs(2), _nbs(1), _nbs(16)],
        out_shape=[jax.ShapeDtypeStruct((NN, 2), F32),
                   jax.ShapeDtypeStruct((NN, 2), F32),
                   jax.ShapeDtypeStruct((NN, 2), F32),
                   jax.ShapeDtypeStruct((NN, 2), F32),
                   jax.ShapeDtypeStruct((NN, 1), F32),
                   jax.ShapeDtypeStruct((NN, 16), F32)],
    )(s0, s1, rec, pos, ab, eps, *small)


def _k_node_halves(sa, sb, rec, pos, ab):
    def body(sa_r, sb_r, rec_r, pos_r, ab_r, h_r):
        mean = jnp.concatenate([sa_r[...], sb_r[...]], axis=1) * rec_r[...]
        h = (mean * ab_r[0:1, :] + ab_r[1:2, :]) * pos_r[...]
        h_r[...] = h.astype(jnp.bfloat16)

    return pl.pallas_call(
        body, grid=_NG,
        in_specs=[_nbs(32), _nbs(32), _nbs(1), _nbs(1), _fix(ab.shape)],
        out_specs=_nbs(64),
        out_shape=jax.ShapeDtypeStruct((NN, 64), jnp.bfloat16),
    )(sa, sb, rec, pos, ab)


def _k_node_final(s0, s1, rec):
    def body(s0_r, s1_r, rec_r, o_r):
        o_r[...] = (s0_r[...] + s1_r[...]) * rec_r[...]

    return pl.pallas_call(
        body, grid=_NG,
        in_specs=[_nbs(16), _nbs(16), _nbs(1)],
        out_specs=_nbs(16),
        out_shape=jax.ShapeDtypeStruct((NN, 16), F32),
    )(s0, s1, rec)


def kernel(x, edge_index, eps, params):
    p = params
    src = edge_index[0]
    dst = edge_index[1]
    padlen = EPAD - EE
    zpad_i = jnp.zeros((padlen,), jnp.int32)
    dst_g = jnp.concatenate([dst, zpad_i]).reshape(EPAD // CH, CH)
    src_g = jnp.concatenate([src, zpad_i]).reshape(EPAD // CH, CH)
    dst_s = jnp.concatenate(
        [dst, jnp.full((padlen,), NN, jnp.int32)]).reshape(EPAD // CH, CH)
    z32 = jnp.zeros((RPT, 32), F32)
    z16 = jnp.zeros((RPT, 16), F32)
    ones16 = jnp.ones((CH, 16), F32)

    cntp = _counts()(dst_s, z16, ones16)
    c0 = cntp[0, :NN, 0:1]
    c1 = cntp[1, :NN, 0:1]

    stx = _k_xstats(x)
    xb = _k_bn0(x, stx, p["bn0"]["g"][None, :], p["bn0"]["b"][None, :])
    xi, xj = _gather2(16, 16)(xb, dst_g, xb, src_g)
    w1 = p["enc1"]["l1"]["W"]
    b1 = p["enc1"]["l1"]["b"][None, :]
    st1 = _k_cat_lin_stats(xi, xj, w1, b1)
    ab1 = _ab(st1, p["enc1"]["bn1"]["g"], p["enc1"]["bn1"]["b"], float(EE))
    y2, st2 = _k_cat_lin_bn_lin(xi, xj, w1, b1, ab1,
                                p["enc1"]["l2"]["W"],
                                p["enc1"]["l2"]["b"][None, :])
    ab2 = _ab(st2, p["enc1"]["bn2"]["g"], p["enc1"]["bn2"]["b"], float(EE))
    y3, st3 = _k_bn_lin_raw(y2, ab2, p["enc1"]["l3"]["W"],
                            p["enc1"]["l3"]["b"][None, :], halves=False)
    ab3 = _ab(st3, p["enc1"]["bn3"]["g"], p["enc1"]["bn3"]["b"], float(EE))
    s1p = _scatter(False, 32)(y3, dst_s, z32)
    h1, rec, pos = _k_node1(s1p[0, :NN], s1p[1, :NN], c0, c1, ab3)

    g2a, g2b = _gather2(32, 32)(h1, dst_g, h1, src_g)
    y1e, st1e = _k_cat_lin(g2a, g2b, p["enc2"]["l1"]["W"],
                           p["enc2"]["l1"]["b"][None, :])
    ab1e = _ab(st1e, p["enc2"]["bn1"]["g"], p["enc2"]["bn1"]["b"], float(EE))
    y2e, st2e = _k_bn_lin_raw(y1e, ab1e, p["enc2"]["l2"]["W"],
                              p["enc2"]["l2"]["b"][None, :], halves=False)
    ab2e = _ab(st2e, p["enc2"]["bn2"]["g"], p["enc2"]["bn2"]["b"], float(EE))
    s2p = _scatter(False, 32)(y2e, dst_s, z32)

    hw = (p["mu"]["W"], p["mu"]["b"][None, :],
          p["var"]["W"], p["var"]["b"][None, :],
          p["amor_d"]["W"], p["amor_d"]["b"][None, :],
          p["amor_diag1"]["W"], p["amor_diag1"]["b"][None, :],
          p["amor_diag2"]["W"], p["amor_diag2"]["b"][None, :],
          p["amor_b"]["W"], p["amor_b"]["b"][None, :])
    mu, log_var, z0, zk, ldj, zkpad = _k_node_mid(
        s2p[0, :NN], s2p[1, :NN], rec, pos, ab2e, eps, hw)

    g3a, g3b = _gather2(16, 16)(zkpad, dst_g, zkpad, src_g)
    w1d = p["dec1"]["l1"]["W"]
    b1d = p["dec1"]["l1"]["b"][None, :]
    w2d = p["dec1"]["l2"]["W"]
    b2d = p["dec1"]["l2"]["b"][None, :]
    st_t2 = _k_d1_stats(g3a, g3b, w1d, b1d, w2d, b2d)
    ab2d = _ab(st_t2, p["dec1"]["bn2"]["g"], p["dec1"]["bn2"]["b"], float(EE))
    y3h, st3d = _k_d1_main(g3a, g3b, w1d, b1d, w2d, b2d, ab2d,
                           p["dec1"]["l3"]["W"], p["dec1"]["l3"]["b"][None, :])
    ab3d = _ab(st3d, p["dec1"]["bn3"]["g"], p["dec1"]["bn3"]["b"], float(EE))
    s3p = _scatter(True, 32)(y3h, dst_s, z32)
    h2 = _k_node_halves(s3p[0, :NN], s3p[1, :NN], rec, pos, ab3d)

    g4a, g4b = _gather2(64, 64, jnp.bfloat16)(h2, dst_g, h2, src_g)
    y1f, st1f = _k_cat_lin(g4a, g4b, p["dec2"]["l1"]["W"],
                           p["dec2"]["l1"]["b"][None, :])
    ab1f = _ab(st1f, p["dec2"]["bn1"]["g"], p["dec2"]["bn1"]["b"], float(EE))
    y2f, st2f = _k_bn_lin_plain(y1f, ab1f, p["dec2"]["l2"]["W"],
                                p["dec2"]["l2"]["b"][None, :])
    ab2f = _ab(st2f, p["dec2"]["bn2"]["g"], p["dec2"]["bn2"]["b"], float(EE))
    m4 = _k_final_edge(y2f, ab2f, p["dec2"]["l3"]["W"],
                       p["dec2"]["l3"]["b"][None, :])
    s4p = _scatter(False, 16)(m4, dst_s, z16)
    x_decoded = _k_node_final(s4p[0, :NN], s4p[1, :NN], rec)

    return (x_decoded, mu, log_var, ldj[:, 0], z0, zk)


# speedup vs baseline: 1.0179x; 1.0072x over previous
import functools

import jax
import jax.numpy as jnp
from jax import lax
from jax.experimental import pallas as pl
from jax.experimental.pallas import tpu as pltpu
from jax.experimental.pallas import tpu_sc as plsc

NN = 50000
EE = 800000
EPAD = 802816
NPAD = 50016
EB = 8192
NBK = 2000
CH = 128
RPT = NPAD // 16
F32 = jnp.float32


def _fix(shape):
    return pl.BlockSpec(shape, lambda i: (0,) * len(shape))


def _ebs(f):
    return pl.BlockSpec((EB, f), lambda i: (i, 0))


def _ebs3(c, f):
    return pl.BlockSpec((c, EB, f), lambda i: (0, i, 0))


def _nbs(f):
    return pl.BlockSpec((NBK, f), lambda i: (i, 0))


def _relu(v):
    return jnp.maximum(v, 0.0)


def _mm(a, b):
    return jnp.dot(a, b, preferred_element_type=F32)


def _cat2(xi, xj):
    return jnp.concatenate([xi, xj - xi], axis=1)


def _acc_stats(i, y, st_ref):
    mask = (i * EB + lax.broadcasted_iota(jnp.int32, (EB, 1), 0)) < EE
    ym = jnp.where(mask, y, 0.0)
    blk = jnp.concatenate(
        [jnp.sum(ym, 0, keepdims=True), jnp.sum(ym * ym, 0, keepdims=True)], 0)

    @pl.when(i == 0)
    def _():
        st_ref[...] = jnp.zeros_like(st_ref)

    st_ref[...] += blk


def _ab(st, g, b, n):
    m = st[0] / n
    v = st[1] / n - m * m
    a = g / jnp.sqrt(v + 1e-5)
    return jnp.stack([a, b - m * a])


def _sc_mesh():
    return plsc.VectorSubcoreMesh(core_axis_name="c", subcore_axis_name="s")


_SC_PARAMS = pltpu.CompilerParams(use_tc_tiling_on_sc=False)


GK = 7
GKS = 7


@functools.lru_cache(maxsize=None)
def _gather2(fa, fb, dt=jnp.float32):
    rows_w = (EPAD // CH) // 32
    ngrp = rows_w // GK

    @functools.partial(
        pl.kernel, mesh=_sc_mesh(), compiler_params=_SC_PARAMS,
        out_type=[jax.ShapeDtypeStruct((EPAD, fa), dt),
                  jax.ShapeDtypeStruct((EPAD, fb), dt)],
        scratch_types=[pltpu.VMEM((GK, CH), jnp.int32),
                       pltpu.VMEM((GK * CH, fa), dt),
                       pltpu.VMEM((GK, CH), jnp.int32),
                       pltpu.VMEM((GK * CH, fb), dt),
                       pltpu.SemaphoreType.DMA,
                       pltpu.SemaphoreType.DMA],
    )
    def k(ta, ia, tb, ib, outa, outb, iva, rva, ivb, rvb, sa, sb):
        wid = lax.axis_index("c") * 16 + lax.axis_index("s")

        def body(g, carry):
            crow = wid * rows_w + g * GK
            base = crow * CH
            pltpu.sync_copy(ia.at[pl.ds(crow, GK)], iva)
            pltpu.sync_copy(ib.at[pl.ds(crow, GK)], ivb)
            cps = []
            for b in range(GK):
                cps.append(pltpu.async_copy(
                    ta.at[iva.at[b]], rva.at[pl.ds(b * CH, CH)], sa))
                cps.append(pltpu.async_copy(
                    tb.at[ivb.at[b]], rvb.at[pl.ds(b * CH, CH)], sb))
            for cp in cps:
                cp.wait()
            pltpu.sync_copy(rva, outa.at[pl.ds(base, GK * CH)])
            pltpu.sync_copy(rvb, outb.at[pl.ds(base, GK * CH)])
            return carry

        lax.fori_loop(0, ngrp, body, 0)

    return k


@functools.lru_cache(maxsize=None)
def _scatter(feature_split, fw):
    if feature_split:
        rows_w = (EPAD // CH) // 16
    else:
        rows_w = (EPAD // CH) // 32
    ngrp = rows_w // GKS

    @functools.partial(
        pl.kernel, mesh=_sc_mesh(), compiler_params=_SC_PARAMS,
        out_type=jax.ShapeDtypeStruct((2, NPAD, fw), F32),
        scratch_types=[pltpu.VMEM((GKS, CH), jnp.int32),
                       pltpu.VMEM((GKS * CH, fw), F32),
                       pltpu.VMEM_SHARED((NPAD, fw), F32),
                       pltpu.SemaphoreType.DMA],
    )
    def k(y, d, z, out, idx_v, row_v, acc, sem):
        cid = lax.axis_index("c")
        sid = lax.axis_index("s")
        pltpu.sync_copy(z, acc.at[pl.ds(sid * RPT, RPT)])
        plsc.subcore_barrier()

        def body(g, carry):
            if feature_split:
                crow = sid * rows_w + g * GKS
                pltpu.sync_copy(y.at[cid, pl.ds(crow * CH, GKS * CH)], row_v)
            else:
                crow = (cid * 16 + sid) * rows_w + g * GKS
                pltpu.sync_copy(y.at[0, pl.ds(crow * CH, GKS * CH)], row_v)
            pltpu.sync_copy(d.at[pl.ds(crow, GKS)], idx_v)
            cps = [pltpu.async_copy(row_v.at[pl.ds(b * CH, CH)],
                                    acc.at[idx_v.at[b]], sem, add=True)
                   for b in range(GKS)]
            for cp in cps:
                cp.wait()
            return carry

        lax.fori_loop(0, ngrp, body, 0)
        plsc.subcore_barrier()
        pltpu.sync_copy(acc.at[pl.ds(sid * RPT, RPT)],
                        out.at[cid, pl.ds(sid * RPT, RPT)])

    return k


@functools.lru_cache(maxsize=None)
def _counts():
    rows_w = (EPAD // CH) // 32
    ngrp = rows_w // GKS

    @functools.partial(
        pl.kernel, mesh=_sc_mesh(), compiler_params=_SC_PARAMS,
        out_type=jax.ShapeDtypeStruct((2, NPAD, 16), F32),
        scratch_types=[pltpu.VMEM((GKS, CH), jnp.int32),
                       pltpu.VMEM((CH, 16), F32),
                       pltpu.VMEM_SHARED((NPAD, 16), F32),
                       pltpu.SemaphoreType.DMA],
    )
    def k(d, z, ones_hbm, out, idx_v, ones_v, acc, sem):
        cid = lax.axis_index("c")
        sid = lax.axis_index("s")
        pltpu.sync_copy(z, acc.at[pl.ds(sid * RPT, RPT)])
        pltpu.sync_copy(ones_hbm, ones_v)
        plsc.subcore_barrier()

        def body(g, carry):
            crow = (cid * 16 + sid) * rows_w + g * GKS
            pltpu.sync_copy(d.at[pl.ds(crow, GKS)], idx_v)
            cps = [pltpu.async_copy(ones_v, acc.at[idx_v.at[b]], sem, add=True)
                   for b in range(GKS)]
            for cp in cps:
                cp.wait()
            return carry

        lax.fori_loop(0, ngrp, body, 0)
        plsc.subcore_barrier()
        pltpu.sync_copy(acc.at[pl.ds(sid * RPT, RPT)],
                        out.at[cid, pl.ds(sid * RPT, RPT)])

    return k


_EG = (EPAD // EB,)


def _k_cat_lin_stats(xi, xj, w1, b1):
    fo = w1.shape[1]

    def body(xi_r, xj_r, w1_r, b1_r, st_r):
        i = pl.program_id(0)
        xiv = xi_r[...].astype(F32)
        xjv = xj_r[...].astype(F32)
        y = _relu(_mm(_cat2(xiv, xjv), w1_r[...]) + b1_r[...])
        _acc_stats(i, y, st_r)

    return pl.pallas_call(
        body, grid=_EG,
        in_specs=[_ebs(xi.shape[1]), _ebs(xj.shape[1]), _fix(w1.shape),
                  _fix(b1.shape)],
        out_specs=_fix((2, fo)),
        out_shape=jax.ShapeDtypeStruct((2, fo), F32),
    )(xi, xj, w1, b1)


def _k_cat_lin_bn_lin(xi, xj, w1, b1, ab, w2, b2):
    fo = w2.shape[1]

    def body(xi_r, xj_r, w1_r, b1_r, ab_r, w2_r, b2_r, y2_r, st_r):
        i = pl.program_id(0)
        y1 = _relu(_mm(_cat2(xi_r[...], xj_r[...]), w1_r[...]) + b1_r[...])
        yh = y1 * ab_r[0:1, :] + ab_r[1:2, :]
        y2 = _relu(_mm(yh, w2_r[...]) + b2_r[...])
        y2_r[...] = y2
        _acc_stats(i, y2, st_r)

    return pl.pallas_call(
        body, grid=_EG,
        in_specs=[_ebs(xi.shape[1]), _ebs(xj.shape[1]), _fix(w1.shape),
                  _fix(b1.shape), _fix(ab.shape), _fix(w2.shape),
                  _fix(b2.shape)],
        out_specs=[_ebs(fo), _fix((2, fo))],
        out_shape=[jax.ShapeDtypeStruct((EPAD, fo), F32),
                   jax.ShapeDtypeStruct((2, fo), F32)],
    )(xi, xj, w1, b1, ab, w2, b2)


def _k_cat_lin(xi, xj, w1, b1):
    fo = w1.shape[1]

    def body(xi_r, xj_r, w1_r, b1_r, y_r, st_r):
        i = pl.program_id(0)
        xiv = xi_r[...].astype(F32)
        xjv = xj_r[...].astype(F32)
        y = _relu(_mm(_cat2(xiv, xjv), w1_r[...]) + b1_r[...])
        y_r[...] = y
        _acc_stats(i, y, st_r)

    return pl.pallas_call(
        body, grid=_EG,
        in_specs=[_ebs(xi.shape[1]), _ebs(xj.shape[1]), _fix(w1.shape),
                  _fix(b1.shape)],
        out_specs=[_ebs(fo), _fix((2, fo))],
        out_shape=[jax.ShapeDtypeStruct((EPAD, fo), F32),
                   jax.ShapeDtypeStruct((2, fo), F32)],
    )(xi, xj, w1, b1)


def _k_bn_lin_raw(y, ab, w, b, halves):
    fo = w.shape[1]

    def body(y_r, ab_r, w_r, b_r, o_r, st_r):
        i = pl.program_id(0)
        yh = y_r[...] * ab_r[0:1, :] + ab_r[1:2, :]
        yo = _relu(_mm(yh, w_r[...]) + b_r[...])
        if halves:
            o_r[0] = yo[:, 0:fo // 2]
            o_r[1] = yo[:, fo // 2:fo]
        else:
            o_r[0] = yo
        _acc_stats(i, yo, st_r)

    c = 2 if halves else 1
    fh = fo // 2 if halves else fo
    return pl.pallas_call(
        body, grid=_EG,
        in_specs=[_ebs(y.shape[1]), _fix(ab.shape), _fix(w.shape), _fix(b.shape)],
        out_specs=[_ebs3(c, fh), _fix((2, fo))],
        out_shape=[jax.ShapeDtypeStruct((c, EPAD, fh), F32),
                   jax.ShapeDtypeStruct((2, fo), F32)],
    )(y, ab, w, b)


def _k_bn_lin_plain(y, ab, w, b):
    fo = w.shape[1]

    def body(y_r, ab_r, w_r, b_r, o_r, st_r):
        i = pl.program_id(0)
        yh = y_r[...] * ab_r[0:1, :] + ab_r[1:2, :]
        yo = _relu(_mm(yh, w_r[...]) + b_r[...])
        o_r[...] = yo
        _acc_stats(i, yo, st_r)

    return pl.pallas_call(
        body, grid=_EG,
        in_specs=[_ebs(y.shape[1]), _fix(ab.shape), _fix(w.shape), _fix(b.shape)],
        out_specs=[_ebs(fo), _fix((2, fo))],
        out_shape=[jax.ShapeDtypeStruct((EPAD, fo), F32),
                   jax.ShapeDtypeStruct((2, fo), F32)],
    )(y, ab, w, b)


def _k_d1_stats(xi, xj, w1, b1, w2, b2):
    def body(xi_r, xj_r, w1_r, b1_r, w2_r, b2_r, st_r):
        i = pl.program_id(0)
        zi = xi_r[...][:, 0:2]
        zj = xj_r[...][:, 0:2]
        y1 = _relu(_mm(_cat2(zi, zj), w1_r[...]) + b1_r[...])
        t2 = _mm(y1, w2_r[...]) + b2_r[...]
        _acc_stats(i, t2, st_r)

    return pl.pallas_call(
        body, grid=_EG,
        in_specs=[_ebs(16), _ebs(16), _fix(w1.shape), _fix(b1.shape),
                  _fix(w2.shape), _fix(b2.shape)],
        out_specs=_fix((2, 32)),
        out_shape=jax.ShapeDtypeStruct((2, 32), F32),
    )(xi, xj, w1, b1, w2, b2)


def _k_d1_main(xi, xj, w1, b1, w2, b2, ab2, w3, b3):
    def body(xi_r, xj_r, w1_r, b1_r, w2_r, b2_r, ab_r, w3_r, b3_r, o_r, st_r):
        i = pl.program_id(0)
        zi = xi_r[...][:, 0:2]
        zj = xj_r[...][:, 0:2]
        y1 = _relu(_mm(_cat2(zi, zj), w1_r[...]) + b1_r[...])
        t2 = _mm(y1, w2_r[...]) + b2_r[...]
        y2 = _relu(t2 * ab_r[0:1, :] + ab_r[1:2, :])
        y3 = _relu(_mm(y2, w3_r[...]) + b3_r[...])
        o_r[0] = y3[:, 0:32]
        o_r[1] = y3[:, 32:64]
        _acc_stats(i, y3, st_r)

    return pl.pallas_call(
        body, grid=_EG,
        in_specs=[_ebs(16), _ebs(16), _fix(w1.shape), _fix(b1.shape),
                  _fix(w2.shape), _fix(b2.shape), _fix(ab2.shape),
                  _fix(w3.shape), _fix(b3.shape)],
        out_specs=[_ebs3(2, 32), _fix((2, 64))],
        out_shape=[jax.ShapeDtypeStruct((2, EPAD, 32), F32),
                   jax.ShapeDtypeStruct((2, 64), F32)],
    )(xi, xj, w1, b1, w2, b2, ab2, w3, b3)


def _k_final_edge(y2, ab2, w3, b3):
    def body(y_r, ab_r, w_r, b_r, o_r):
        yh = y_r[...] * ab_r[0:1, :] + ab_r[1:2, :]
        o_r[0] = _mm(yh, w_r[...]) + b_r[...]

    return pl.pallas_call(
        body, grid=_EG,
        in_specs=[_ebs(y2.shape[1]), _fix(ab2.shape), _fix(w3.shape),
                  _fix(b3.shape)],
        out_specs=_ebs3(1, 16),
        out_shape=jax.ShapeDtypeStruct((1, EPAD, 16), F32),
    )(y2, ab2, w3, b3)


_NG = (NN // NBK,)


def _k_xstats(x):
    def body(x_r, st_r):
        i = pl.program_id(0)
        xv = x_r[...]
        blk = jnp.concatenate(
            [jnp.sum(xv, 0, keepdims=True), jnp.sum(xv * xv, 0, keepdims=True)], 0)

        @pl.when(i == 0)
        def _():
            st_r[...] = jnp.zeros_like(st_r)

        st_r[...] += blk

    return pl.pallas_call(
        body, grid=_NG, in_specs=[_nbs(16)], out_specs=_fix((2, 16)),
        out_shape=jax.ShapeDtypeStruct((2, 16), F32))(x)


def _k_bn0(x, st, g, b):
    def body(x_r, st_r, g_r, b_r, o_r):
        m = st_r[0:1, :] / float(NN)
        v = st_r[1:2, :] / float(NN) - m * m
        o_r[...] = g_r[...] * (x_r[...] - m) / jnp.sqrt(v + 1e-5) + b_r[...]

    return pl.pallas_call(
        body, grid=_NG,
        in_specs=[_nbs(16), _fix((2, 16)), _fix((1, 16)), _fix((1, 16))],
        out_specs=_nbs(16),
        out_shape=jax.ShapeDtypeStruct((NN, 16), F32))(x, st, g, b)


def _k_node1(s0, s1, c0, c1, ab):
    f = s0.shape[1]

    def body(s0_r, s1_r, c0_r, c1_r, ab_r, h_r, rec_r, pos_r):
        cnt = c0_r[...] + c1_r[...]
        rec = 1.0 / jnp.maximum(cnt, 1.0)
        pos = jnp.where(cnt > 0.0, 1.0, 0.0)
        mean = (s0_r[...] + s1_r[...]) * rec
        h_r[...] = (mean * ab_r[0:1, :] + ab_r[1:2, :]) * pos
        rec_r[...] = rec
        pos_r[...] = pos

    return pl.pallas_call(
        body, grid=_NG,
        in_specs=[_nbs(f), _nbs(f), _nbs(1), _nbs(1), _fix(ab.shape)],
        out_specs=[_nbs(f), _nbs(1), _nbs(1)],
        out_shape=[jax.ShapeDtypeStruct((NN, f), F32),
                   jax.ShapeDtypeStruct((NN, 1), F32),
                   jax.ShapeDtypeStruct((NN, 1), F32)],
    )(s0, s1, c0, c1, ab)


def _k_node_mid(s0, s1, rec, pos, ab, eps, hw):
    (wmu, bmu, wvar, bvar, wd, bd, wd1, bd1, wd2, bd2, wbf, bbf) = hw

    def body(s0_r, s1_r, rec_r, pos_r, ab_r, eps_r, wmu_r, bmu_r, wvar_r,
             bvar_r, wd_r, bd_r, wd1_r, bd1_r, wd2_r, bd2_r, wbf_r, bbf_r,
             mu_r, lv_r, z0_r, zk_r, ldj_r, zp_r):
        mean = (s0_r[...] + s1_r[...]) * rec_r[...]
        h = (mean * ab_r[0:1, :] + ab_r[1:2, :]) * pos_r[...]
        mu = _mm(h, wmu_r[...]) + bmu_r[...]
        lv = _mm(h, wvar_r[...]) + bvar_r[...]
        fd = _mm(h, wd_r[...]) + bd_r[...]
        d1 = jnp.tanh(_mm(h, wd1_r[...]) + bd1_r[...])
        d2 = jnp.tanh(_mm(h, wd2_r[...]) + bd2_r[...])
        bf = _mm(h, wbf_r[...]) + bbf_r[...]
        z0 = mu + eps_r[...] * jnp.exp(0.5 * lv)
        zc0 = z0[:, 0:1]
        zc1 = z0[:, 1:2]
        ldj = jnp.zeros_like(zc0)
        for k in range(6):
            fd01 = fd[:, 6 + k:7 + k]
            fd10 = fd[:, 12 + k:13 + k]
            d1_0 = d1[:, k:k + 1]
            d1_1 = d1[:, 6 + k:7 + k]
            d2_0 = d2[:, k:k + 1]
            d2_1 = d2[:, 6 + k:7 + k]
            b_0 = bf[:, k:k + 1]
            b_1 = bf[:, 6 + k:7 + k]
            if k % 2 == 1:
                zp0, zp1 = zc1, zc0
            else:
                zp0, zp1 = zc0, zc1
            t0 = jnp.tanh(zp0 * d2_0 + zp1 * fd10 + b_0)
            t1 = jnp.tanh(zp1 * d2_1 + b_1)
            n0 = t0 * d1_0 + t1 * fd01
            n1 = t1 * d1_1
            if k % 2 == 1:
                n0, n1 = n1, n0
            zc0 = zc0 + n0
            zc1 = zc1 + n1
            dj0 = (1.0 - t0 * t0) * d1_0 * d2_0 + 1.0
            dj1 = (1.0 - t1 * t1) * d1_1 * d2_1 + 1.0
            ldj = ldj + jnp.log(jnp.abs(dj0)) + jnp.log(jnp.abs(dj1))
        zk = jnp.concatenate([zc0, zc1], axis=1)
        mu_r[...] = mu
        lv_r[...] = lv
        z0_r[...] = z0
        zk_r[...] = zk
        ldj_r[...] = ldj
        zp_r[...] = jnp.concatenate(
            [zk, jnp.zeros((zk.shape[0], 14), F32)], axis=1)

    small = [wmu, bmu, wvar, bvar, wd, bd, wd1, bd1, wd2, bd2, wbf, bbf]
    return pl.pallas_call(
        body, grid=_NG,
        in_specs=[_nbs(32), _nbs(32), _nbs(1), _nbs(1), _fix(ab.shape),
                  _nbs(2)] + [_fix(a.shape) for a in small],
        out_specs=[_nbs(2), _nbs(2), _nbs(2), _nbs(2), _nbs(1), _nbs(16)],
        out_shape=[jax.ShapeDtypeStruct((NN, 2), F32),
                   jax.ShapeDtypeStruct((NN, 2), F32),
                   jax.ShapeDtypeStruct((NN, 2), F32),
                   jax.ShapeDtypeStruct((NN, 2), F32),
                   jax.ShapeDtypeStruct((NN, 1), F32),
                   jax.ShapeDtypeStruct((NN, 16), F32)],
    )(s0, s1, rec, pos, ab, eps, *small)


def _k_node_halves(sa, sb, rec, pos, ab):
    def body(sa_r, sb_r, rec_r, pos_r, ab_r, h_r):
        mean = jnp.concatenate([sa_r[...], sb_r[...]], axis=1) * rec_r[...]
        h_r[...] = (mean * ab_r[0:1, :] + ab_r[1:2, :]) * pos_r[...]

    return pl.pallas_call(
        body, grid=_NG,
        in_specs=[_nbs(32), _nbs(32), _nbs(1), _nbs(1), _fix(ab.shape)],
        out_specs=_nbs(64),
        out_shape=jax.ShapeDtypeStruct((NN, 64), F32),
    )(sa, sb, rec, pos, ab)


def _k_node_final(s0, s1, rec):
    def body(s0_r, s1_r, rec_r, o_r):
        o_r[...] = (s0_r[...] + s1_r[...]) * rec_r[...]

    return pl.pallas_call(
        body, grid=_NG,
        in_specs=[_nbs(16), _nbs(16), _nbs(1)],
        out_specs=_nbs(16),
        out_shape=jax.ShapeDtypeStruct((NN, 16), F32),
    )(s0, s1, rec)


def kernel(x, edge_index, eps, params):
    p = params
    src = edge_index[0]
    dst = edge_index[1]
    padlen = EPAD - EE
    zpad_i = jnp.zeros((padlen,), jnp.int32)
    dst_g = jnp.concatenate([dst, zpad_i]).reshape(EPAD // CH, CH)
    src_g = jnp.concatenate([src, zpad_i]).reshape(EPAD // CH, CH)
    dst_s = jnp.concatenate(
        [dst, jnp.full((padlen,), NN, jnp.int32)]).reshape(EPAD // CH, CH)
    z32 = jnp.zeros((RPT, 32), F32)
    z16 = jnp.zeros((RPT, 16), F32)
    ones16 = jnp.ones((CH, 16), F32)

    cntp = _counts()(dst_s, z16, ones16)
    c0 = cntp[0, :NN, 0:1]
    c1 = cntp[1, :NN, 0:1]

    stx = _k_xstats(x)
    xb = _k_bn0(x, stx, p["bn0"]["g"][None, :], p["bn0"]["b"][None, :])
    xi, xj = _gather2(16, 16)(xb, dst_g, xb, src_g)
    w1 = p["enc1"]["l1"]["W"]
    b1 = p["enc1"]["l1"]["b"][None, :]
    st1 = _k_cat_lin_stats(xi, xj, w1, b1)
    ab1 = _ab(st1, p["enc1"]["bn1"]["g"], p["enc1"]["bn1"]["b"], float(EE))
    y2, st2 = _k_cat_lin_bn_lin(xi, xj, w1, b1, ab1,
                                p["enc1"]["l2"]["W"],
                                p["enc1"]["l2"]["b"][None, :])
    ab2 = _ab(st2, p["enc1"]["bn2"]["g"], p["enc1"]["bn2"]["b"], float(EE))
    y3, st3 = _k_bn_lin_raw(y2, ab2, p["enc1"]["l3"]["W"],
                            p["enc1"]["l3"]["b"][None, :], halves=False)
    ab3 = _ab(st3, p["enc1"]["bn3"]["g"], p["enc1"]["bn3"]["b"], float(EE))
    s1p = _scatter(False, 32)(y3, dst_s, z32)
    h1, rec, pos = _k_node1(s1p[0, :NN], s1p[1, :NN], c0, c1, ab3)

    g2a, g2b = _gather2(32, 32)(h1, dst_g, h1, src_g)
    y1e, st1e = _k_cat_lin(g2a, g2b, p["enc2"]["l1"]["W"],
                           p["enc2"]["l1"]["b"][None, :])
    ab1e = _ab(st1e, p["enc2"]["bn1"]["g"], p["enc2"]["bn1"]["b"], float(EE))
    y2e, st2e = _k_bn_lin_raw(y1e, ab1e, p["enc2"]["l2"]["W"],
                              p["enc2"]["l2"]["b"][None, :], halves=False)
    ab2e = _ab(st2e, p["enc2"]["bn2"]["g"], p["enc2"]["bn2"]["b"], float(EE))
    s2p = _scatter(False, 32)(y2e, dst_s, z32)

    hw = (p["mu"]["W"], p["mu"]["b"][None, :],
          p["var"]["W"], p["var"]["b"][None, :],
          p["amor_d"]["W"], p["amor_d"]["b"][None, :],
          p["amor_diag1"]["W"], p["amor_diag1"]["b"][None, :],
          p["amor_diag2"]["W"], p["amor_diag2"]["b"][None, :],
          p["amor_b"]["W"], p["amor_b"]["b"][None, :])
    mu, log_var, z0, zk, ldj, zkpad = _k_node_mid(
        s2p[0, :NN], s2p[1, :NN], rec, pos, ab2e, eps, hw)

    g3a, g3b = _gather2(16, 16)(zkpad, dst_g, zkpad, src_g)
    w1d = p["dec1"]["l1"]["W"]
    b1d = p["dec1"]["l1"]["b"][None, :]
    w2d = p["dec1"]["l2"]["W"]
    b2d = p["dec1"]["l2"]["b"][None, :]
    st_t2 = _k_d1_stats(g3a, g3b, w1d, b1d, w2d, b2d)
    ab2d = _ab(st_t2, p["dec1"]["bn2"]["g"], p["dec1"]["bn2"]["b"], float(EE))
    y3h, st3d = _k_d1_main(g3a, g3b, w1d, b1d, w2d, b2d, ab2d,
                           p["dec1"]["l3"]["W"], p["dec1"]["l3"]["b"][None, :])
    ab3d = _ab(st3d, p["dec1"]["bn3"]["g"], p["dec1"]["bn3"]["b"], float(EE))
    s3p = _scatter(True, 32)(y3h, dst_s, z32)
    h2 = _k_node_halves(s3p[0, :NN], s3p[1, :NN], rec, pos, ab3d)

    g4a, g4b = _gather2(64, 64)(h2, dst_g, h2, src_g)
    y1f, st1f = _k_cat_lin(g4a, g4b, p["dec2"]["l1"]["W"],
                           p["dec2"]["l1"]["b"][None, :])
    ab1f = _ab(st1f, p["dec2"]["bn1"]["g"], p["dec2"]["bn1"]["b"], float(EE))
    y2f, st2f = _k_bn_lin_plain(y1f, ab1f, p["dec2"]["l2"]["W"],
                                p["dec2"]["l2"]["b"][None, :])
    ab2f = _ab(st2f, p["dec2"]["bn2"]["g"], p["dec2"]["bn2"]["b"], float(EE))
    m4 = _k_final_edge(y2f, ab2f, p["dec2"]["l3"]["W"],
                       p["dec2"]["l3"]["b"][None, :])
    s4p = _scatter(False, 16)(m4, dst_s, z16)
    x_decoded = _k_node_final(s4p[0, :NN], s4p[1, :NN], rec)

    return (x_decoded, mu, log_var, ldj[:, 0], z0, zk)
